# baseline (speedup 1.0000x reference)
import functools

import jax
import jax.numpy as jnp
from jax import lax
from jax.experimental import pallas as pl
from jax.experimental.pallas import tpu as pltpu
from jax.experimental.pallas import tpu_sc as plsc

B, T, C = 4, 2048, 384
H, D = 6, 64
E = 8
FF = 4 * C
N = B * T

BQ = 512
BM = 256
G = N // BM + E
P = G * BM

_NEG = -1e30


def _ln(xx, g, b):
    m = jnp.mean(xx, axis=-1, keepdims=True)
    v = jnp.mean((xx - m) ** 2, axis=-1, keepdims=True)
    return (xx - m) * lax.rsqrt(v + 1e-5) * g + b


def _qkv_body(x_ref, g_ref, b_ref, w_ref, q_ref, k_ref, v_ref):
    hh = _ln(x_ref[0], g_ref[...], b_ref[...])
    qkv = jnp.dot(hh, w_ref[...], preferred_element_type=jnp.float32)
    for h in range(H):
        q_ref[0, h] = qkv[:, h * D:(h + 1) * D]
        k_ref[0, h] = qkv[:, (H + h) * D:(H + h + 1) * D]
        v_ref[0, h] = qkv[:, (2 * H + h) * D:(2 * H + h + 1) * D]


def _qkv(x, ln1_g, ln1_b, Wq, Wk, Wv):
    Wqkv = jnp.concatenate([
        Wq.transpose(1, 0, 2).reshape(C, H * D),
        Wk.transpose(1, 0, 2).reshape(C, H * D),
        Wv.transpose(1, 0, 2).reshape(C, H * D),
    ], axis=1)
    out = jax.ShapeDtypeStruct((B, H, T, D), jnp.float32)
    return pl.pallas_call(
        _qkv_body,
        grid=(B,),
        in_specs=[
            pl.BlockSpec((1, T, C), lambda b: (b, 0, 0)),
            pl.BlockSpec((1, C), lambda b: (0, 0)),
            pl.BlockSpec((1, C), lambda b: (0, 0)),
            pl.BlockSpec((C, 3 * H * D), lambda b: (0, 0)),
        ],
        out_specs=[
            pl.BlockSpec((1, H, T, D), lambda b: (b, 0, 0, 0)),
            pl.BlockSpec((1, H, T, D), lambda b: (b, 0, 0, 0)),
            pl.BlockSpec((1, H, T, D), lambda b: (b, 0, 0, 0)),
        ],
        out_shape=[out, out, out],
    )(x, ln1_g.reshape(1, C), ln1_b.reshape(1, C), Wqkv)


def _attn_body(q_ref, k_ref, v_ref, o_ref):
    i = pl.program_id(2)
    q = q_ref[0, 0]
    row = i * BQ + lax.broadcasted_iota(jnp.int32, (BQ, BQ), 0)
    col0 = lax.broadcasted_iota(jnp.int32, (BQ, BQ), 1)

    def body(j, carry):
        m, l, acc = carry
        kj = k_ref[0, 0, pl.ds(j * BQ, BQ), :]
        s = lax.dot_general(q, kj, (((1,), (1,)), ((), ())),
                            preferred_element_type=jnp.float32)
        s = jnp.where(j * BQ + col0 <= row, s * (C ** -0.5), _NEG)
        mn = jnp.maximum(m, jnp.max(s, axis=-1, keepdims=True))
        p = jnp.exp(s - mn)
        alpha = jnp.exp(m - mn)
        l = l * alpha + jnp.sum(p, axis=-1, keepdims=True)
        vj = v_ref[0, 0, pl.ds(j * BQ, BQ), :]
        acc = acc * alpha + jnp.dot(p, vj, preferred_element_type=jnp.float32)
        return mn, l, acc

    m0 = jnp.full((BQ, 1), _NEG, jnp.float32)
    l0 = jnp.zeros((BQ, 1), jnp.float32)
    a0 = jnp.zeros((BQ, D), jnp.float32)
    m, l, acc = lax.fori_loop(0, i + 1, body, (m0, l0, a0))
    o_ref[0, 0] = acc / l


def _attn(q, k, v):
    return pl.pallas_call(
        _attn_body,
        grid=(B, H, T // BQ),
        in_specs=[
            pl.BlockSpec((1, 1, BQ, D), lambda b, h, i: (b, h, i, 0)),
            pl.BlockSpec((1, 1, T, D), lambda b, h, i: (b, h, 0, 0)),
            pl.BlockSpec((1, 1, T, D), lambda b, h, i: (b, h, 0, 0)),
        ],
        out_specs=pl.BlockSpec((1, 1, BQ, D), lambda b, h, i: (b, h, i, 0)),
        out_shape=jax.ShapeDtypeStruct((B, H, T, D), jnp.float32),
    )(q, k, v)


def _post_body(x_ref, att_ref, wp_ref, bp_ref, g_ref, b_ref, wg_ref,
               x1_ref, h2_ref, sel_ref):
    acc = x_ref[0] + bp_ref[...]
    for h in range(H):
        acc = acc + jnp.dot(att_ref[0, h], wp_ref[h * D:(h + 1) * D, :],
                            preferred_element_type=jnp.float32)
    x1_ref[0] = acc
    h2 = _ln(acc, g_ref[...], b_ref[...])
    h2_ref[0] = h2
    logits = jnp.dot(h2, wg_ref[...], preferred_element_type=jnp.float32)
    best_v = logits[:, 0:1]
    best_i = jnp.zeros((T, 1), jnp.int32)
    for e in range(1, E):
        ve = logits[:, e:e + 1]
        upd = ve > best_v
        best_v = jnp.where(upd, ve, best_v)
        best_i = jnp.where(upd, e, best_i)
    sel_ref[0] = best_i


def _post(x, att, Wp, bp, ln2_g, ln2_b, Wg):
    return pl.pallas_call(
        _post_body,
        grid=(B,),
        in_specs=[
            pl.BlockSpec((1, T, C), lambda b: (b, 0, 0)),
            pl.BlockSpec((1, H, T, D), lambda b: (b, 0, 0, 0)),
            pl.BlockSpec((C, C), lambda b: (0, 0)),
            pl.BlockSpec((1, C), lambda b: (0, 0)),
            pl.BlockSpec((1, C), lambda b: (0, 0)),
            pl.BlockSpec((1, C), lambda b: (0, 0)),
            pl.BlockSpec((C, E), lambda b: (0, 0)),
        ],
        out_specs=[
            pl.BlockSpec((1, T, C), lambda b: (b, 0, 0)),
            pl.BlockSpec((1, T, C), lambda b: (b, 0, 0)),
            pl.BlockSpec((1, T, 1), lambda b: (b, 0, 0)),
        ],
        out_shape=[
            jax.ShapeDtypeStruct((B, T, C), jnp.float32),
            jax.ShapeDtypeStruct((B, T, C), jnp.float32),
            jax.ShapeDtypeStruct((B, T, 1), jnp.int32),
        ],
    )(x, att, Wp, bp.reshape(1, C), ln2_g.reshape(1, C), ln2_b.reshape(1, C), Wg)


def _sc_gather(table, idx, rows, chunk):
    nw = 32
    b_per_w = rows // nw
    n_ch = b_per_w // chunk
    mesh = plsc.VectorSubcoreMesh(core_axis_name="c", subcore_axis_name="s")

    @functools.partial(
        pl.kernel,
        out_type=jax.ShapeDtypeStruct((rows, C), jnp.float32),
        mesh=mesh,
        scratch_types=[
            pltpu.VMEM((chunk,), jnp.int32),
            pltpu.VMEM((chunk, C), jnp.float32),
            pltpu.SemaphoreType.DMA,
        ],
    )
    def k(table_hbm, idx_hbm, out_hbm, idx_v, rows_v, sem):
        wid = lax.axis_index("s") * 2 + lax.axis_index("c")
        base0 = wid * b_per_w
        for i in range(n_ch):
            base = base0 + i * chunk
            pltpu.sync_copy(idx_hbm.at[pl.ds(base, chunk)], idx_v)
            pltpu.async_copy(table_hbm.at[idx_v], rows_v, sem).wait()
            pltpu.sync_copy(rows_v, out_hbm.at[pl.ds(base, chunk)])

    return k(table, idx)


def _gmm_body(be_ref, bv_ref, xs_ref, w1_ref, b1_ref, w2_ref, b2_ref, out_ref):
    i = pl.program_id(0)

    @pl.when(bv_ref[i] == 1)
    def _():
        e = be_ref[i]
        a = jnp.dot(xs_ref[...].astype(jnp.bfloat16), w1_ref[e],
                    preferred_element_type=jnp.float32)
        a = jnp.maximum(a + b1_ref[e], 0.0)
        out_ref[...] = (jnp.dot(a.astype(jnp.bfloat16), w2_ref[e],
                                preferred_element_type=jnp.float32)
                        + b2_ref[e])


def _gmm(bexp, bvalid, xs, We1, be1, We2, be2):
    grid_spec = pltpu.PrefetchScalarGridSpec(
        num_scalar_prefetch=2,
        grid=(G,),
        in_specs=[
            pl.BlockSpec((BM, C), lambda i, be, bv: (i, 0)),
            pl.BlockSpec((E, C, FF), lambda i, be, bv: (0, 0, 0)),
            pl.BlockSpec((E, 1, FF), lambda i, be, bv: (0, 0, 0)),
            pl.BlockSpec((E, FF, C), lambda i, be, bv: (0, 0, 0)),
            pl.BlockSpec((E, 1, C), lambda i, be, bv: (0, 0, 0)),
        ],
        out_specs=pl.BlockSpec((BM, C), lambda i, be, bv: (i, 0)),
    )
    return pl.pallas_call(
        _gmm_body,
        grid_spec=grid_spec,
        out_shape=jax.ShapeDtypeStruct((P, C), jnp.float32),
    )(bexp, bvalid, xs,
      We1.astype(jnp.bfloat16), be1.reshape(E, 1, FF),
      We2.astype(jnp.bfloat16), be2.reshape(E, 1, C))


def _add_body(a_ref, b_ref, o_ref):
    o_ref[...] = a_ref[...] + b_ref[...]


def _add(a, b):
    return pl.pallas_call(
        _add_body,
        grid=(B,),
        in_specs=[
            pl.BlockSpec((1, T, C), lambda i: (i, 0, 0)),
            pl.BlockSpec((1, T, C), lambda i: (i, 0, 0)),
        ],
        out_specs=pl.BlockSpec((1, T, C), lambda i: (i, 0, 0)),
        out_shape=jax.ShapeDtypeStruct((B, T, C), jnp.float32),
    )(a, b)


def kernel(x, ln1_g, ln1_b, Wq, Wk, Wv, Wp, bp, ln2_g, ln2_b, Wg, We1, be1, We2, be2):
    q, k, v = _qkv(x, ln1_g, ln1_b, Wq, Wk, Wv)
    att = _attn(q, k, v)
    x1, h2, sel3 = _post(x, att, Wp, bp, ln2_g, ln2_b, Wg)

    sel = sel3.reshape(N)
    perm = jnp.argsort(sel).astype(jnp.int32)
    sorted_e = sel[perm]
    counts = jnp.zeros((E,), jnp.int32).at[sel].add(1)
    off = jnp.concatenate([jnp.zeros((1,), jnp.int32),
                           jnp.cumsum(counts)[:-1].astype(jnp.int32)])
    pc = ((counts + BM - 1) // BM) * BM
    blocks_cum = jnp.cumsum(pc // BM).astype(jnp.int32)
    start_p = jnp.concatenate([jnp.zeros((1,), jnp.int32),
                               (jnp.cumsum(pc)[:-1]).astype(jnp.int32)])
    g_used = blocks_cum[-1]
    gar = jnp.arange(G, dtype=jnp.int32)
    bexp = jnp.searchsorted(blocks_cum, gar, side='right').astype(jnp.int32)
    bvalid = (gar < g_used).astype(jnp.int32)
    last_e = bexp[jnp.maximum(g_used - 1, 0)]
    bexp = jnp.where(bvalid == 1, bexp, last_e)
    within = jnp.arange(N, dtype=jnp.int32) - off[sorted_e]
    ppos_sorted = start_p[sorted_e] + within
    ridx = (jnp.arange(P, dtype=jnp.int32) % N).at[ppos_sorted].set(perm)
    ppos = jnp.zeros((N,), jnp.int32).at[perm].set(ppos_sorted)

    xs = _sc_gather(h2.reshape(N, C), ridx, P, P // 32)
    ys = _gmm(bexp, bvalid, xs, We1, be1, We2, be2)
    res = _sc_gather(ys, ppos, N, N // 32)
    return _add(x1, res.reshape(B, T, C))


# speedup vs baseline: 1.1381x; 1.1381x over previous
import functools

import jax
import jax.numpy as jnp
from jax import lax
from jax.experimental import pallas as pl
from jax.experimental.pallas import tpu as pltpu
from jax.experimental.pallas import tpu_sc as plsc

B, T, C = 4, 2048, 384
H, D = 6, 64
E = 8
FF = 4 * C
N = B * T

BQ = 512
BM = 256
G = N // BM + E
P = G * BM

_NEG = -1e30


def _ln(xx, g, b):
    m = jnp.mean(xx, axis=-1, keepdims=True)
    v = jnp.mean((xx - m) ** 2, axis=-1, keepdims=True)
    return (xx - m) * lax.rsqrt(v + 1e-5) * g + b


def _qkv_body(x_ref, g_ref, b_ref, w_ref, q_ref, k_ref, v_ref):
    hh = _ln(x_ref[0], g_ref[...], b_ref[...])
    qkv = jnp.dot(hh.astype(jnp.bfloat16), w_ref[...],
                  preferred_element_type=jnp.float32).astype(jnp.bfloat16)
    for h in range(H):
        q_ref[0, h] = qkv[:, h * D:(h + 1) * D]
        k_ref[0, h] = qkv[:, (H + h) * D:(H + h + 1) * D]
        v_ref[0, h] = qkv[:, (2 * H + h) * D:(2 * H + h + 1) * D]


def _qkv(x, ln1_g, ln1_b, Wq, Wk, Wv):
    Wqkv = jnp.concatenate([
        Wq.transpose(1, 0, 2).reshape(C, H * D),
        Wk.transpose(1, 0, 2).reshape(C, H * D),
        Wv.transpose(1, 0, 2).reshape(C, H * D),
    ], axis=1).astype(jnp.bfloat16)
    out = jax.ShapeDtypeStruct((B, H, T, D), jnp.bfloat16)
    return pl.pallas_call(
        _qkv_body,
        grid=(B,),
        in_specs=[
            pl.BlockSpec((1, T, C), lambda b: (b, 0, 0)),
            pl.BlockSpec((1, C), lambda b: (0, 0)),
            pl.BlockSpec((1, C), lambda b: (0, 0)),
            pl.BlockSpec((C, 3 * H * D), lambda b: (0, 0)),
        ],
        out_specs=[
            pl.BlockSpec((1, H, T, D), lambda b: (b, 0, 0, 0)),
            pl.BlockSpec((1, H, T, D), lambda b: (b, 0, 0, 0)),
            pl.BlockSpec((1, H, T, D), lambda b: (b, 0, 0, 0)),
        ],
        out_shape=[out, out, out],
    )(x, ln1_g.reshape(1, C), ln1_b.reshape(1, C), Wqkv)


def _attn_body(q_ref, k_ref, v_ref, o_ref):
    i = pl.program_id(2)
    q = q_ref[0, 0]
    row = i * BQ + lax.broadcasted_iota(jnp.int32, (BQ, BQ), 0)
    col0 = lax.broadcasted_iota(jnp.int32, (BQ, BQ), 1)

    def body(j, carry):
        m, l, acc = carry
        kj = k_ref[0, 0, pl.ds(j * BQ, BQ), :]
        s = lax.dot_general(q, kj, (((1,), (1,)), ((), ())),
                            preferred_element_type=jnp.float32)
        s = jnp.where(j * BQ + col0 <= row, s * (C ** -0.5), _NEG)
        mn = jnp.maximum(m, jnp.max(s, axis=-1, keepdims=True))
        p = jnp.exp(s - mn)
        alpha = jnp.exp(m - mn)
        l = l * alpha + jnp.sum(p, axis=-1, keepdims=True)
        vj = v_ref[0, 0, pl.ds(j * BQ, BQ), :]
        acc = acc * alpha + jnp.dot(p.astype(jnp.bfloat16), vj,
                                    preferred_element_type=jnp.float32)
        return mn, l, acc

    m0 = jnp.full((BQ, 1), _NEG, jnp.float32)
    l0 = jnp.zeros((BQ, 1), jnp.float32)
    a0 = jnp.zeros((BQ, D), jnp.float32)
    m, l, acc = lax.fori_loop(0, i + 1, body, (m0, l0, a0))
    o_ref[0, 0] = (acc / l).astype(jnp.bfloat16)


def _attn(q, k, v):
    return pl.pallas_call(
        _attn_body,
        grid=(B, H, T // BQ),
        in_specs=[
            pl.BlockSpec((1, 1, BQ, D), lambda b, h, i: (b, h, i, 0)),
            pl.BlockSpec((1, 1, T, D), lambda b, h, i: (b, h, 0, 0)),
            pl.BlockSpec((1, 1, T, D), lambda b, h, i: (b, h, 0, 0)),
        ],
        out_specs=pl.BlockSpec((1, 1, BQ, D), lambda b, h, i: (b, h, i, 0)),
        out_shape=jax.ShapeDtypeStruct((B, H, T, D), jnp.bfloat16),
    )(q, k, v)


def _post_body(x_ref, att_ref, wp_ref, bp_ref, g_ref, b_ref, wg_ref,
               x1_ref, h2_ref, sel_ref):
    acc = x_ref[0] + bp_ref[...]
    for h in range(H):
        acc = acc + jnp.dot(att_ref[0, h],
                            wp_ref[h * D:(h + 1) * D, :].astype(jnp.bfloat16),
                            preferred_element_type=jnp.float32)
    x1_ref[0] = acc
    h2 = _ln(acc, g_ref[...], b_ref[...])
    h2_ref[0] = h2
    logits = jnp.dot(h2, wg_ref[...], preferred_element_type=jnp.float32)
    best_v = logits[:, 0:1]
    best_i = jnp.zeros((T, 1), jnp.int32)
    for e in range(1, E):
        ve = logits[:, e:e + 1]
        upd = ve > best_v
        best_v = jnp.where(upd, ve, best_v)
        best_i = jnp.where(upd, e, best_i)
    sel_ref[0] = best_i


def _post(x, att, Wp, bp, ln2_g, ln2_b, Wg):
    return pl.pallas_call(
        _post_body,
        grid=(B,),
        in_specs=[
            pl.BlockSpec((1, T, C), lambda b: (b, 0, 0)),
            pl.BlockSpec((1, H, T, D), lambda b: (b, 0, 0, 0)),
            pl.BlockSpec((C, C), lambda b: (0, 0)),
            pl.BlockSpec((1, C), lambda b: (0, 0)),
            pl.BlockSpec((1, C), lambda b: (0, 0)),
            pl.BlockSpec((1, C), lambda b: (0, 0)),
            pl.BlockSpec((C, E), lambda b: (0, 0)),
        ],
        out_specs=[
            pl.BlockSpec((1, T, C), lambda b: (b, 0, 0)),
            pl.BlockSpec((1, T, C), lambda b: (b, 0, 0)),
            pl.BlockSpec((1, T, 1), lambda b: (b, 0, 0)),
        ],
        out_shape=[
            jax.ShapeDtypeStruct((B, T, C), jnp.float32),
            jax.ShapeDtypeStruct((B, T, C), jnp.float32),
            jax.ShapeDtypeStruct((B, T, 1), jnp.int32),
        ],
    )(x, att, Wp, bp.reshape(1, C), ln2_g.reshape(1, C), ln2_b.reshape(1, C), Wg)


def _sc_gather(table, idx, rows, chunk):
    nw = 32
    b_per_w = rows // nw
    n_ch = b_per_w // chunk
    mesh = plsc.VectorSubcoreMesh(core_axis_name="c", subcore_axis_name="s")

    @functools.partial(
        pl.kernel,
        out_type=jax.ShapeDtypeStruct((rows, C), jnp.float32),
        mesh=mesh,
        scratch_types=[
            pltpu.VMEM((chunk,), jnp.int32),
            pltpu.VMEM((chunk, C), jnp.float32),
            pltpu.SemaphoreType.DMA,
        ],
    )
    def k(table_hbm, idx_hbm, out_hbm, idx_v, rows_v, sem):
        wid = lax.axis_index("s") * 2 + lax.axis_index("c")
        base0 = wid * b_per_w
        for i in range(n_ch):
            base = base0 + i * chunk
            pltpu.sync_copy(idx_hbm.at[pl.ds(base, chunk)], idx_v)
            pltpu.async_copy(table_hbm.at[idx_v], rows_v, sem).wait()
            pltpu.sync_copy(rows_v, out_hbm.at[pl.ds(base, chunk)])

    return k(table, idx)


def _gmm_body(be_ref, bv_ref, xs_ref, w1_ref, b1_ref, w2_ref, b2_ref, out_ref):
    i = pl.program_id(0)

    @pl.when(bv_ref[i] == 1)
    def _():
        e = be_ref[i]
        a = jnp.dot(xs_ref[...].astype(jnp.bfloat16), w1_ref[e],
                    preferred_element_type=jnp.float32)
        a = jnp.maximum(a + b1_ref[e], 0.0)
        out_ref[...] = (jnp.dot(a.astype(jnp.bfloat16), w2_ref[e],
                                preferred_element_type=jnp.float32)
                        + b2_ref[e])


def _gmm(bexp, bvalid, xs, We1, be1, We2, be2):
    grid_spec = pltpu.PrefetchScalarGridSpec(
        num_scalar_prefetch=2,
        grid=(G,),
        in_specs=[
            pl.BlockSpec((BM, C), lambda i, be, bv: (i, 0)),
            pl.BlockSpec((E, C, FF), lambda i, be, bv: (0, 0, 0)),
            pl.BlockSpec((E, 1, FF), lambda i, be, bv: (0, 0, 0)),
            pl.BlockSpec((E, FF, C), lambda i, be, bv: (0, 0, 0)),
            pl.BlockSpec((E, 1, C), lambda i, be, bv: (0, 0, 0)),
        ],
        out_specs=pl.BlockSpec((BM, C), lambda i, be, bv: (i, 0)),
    )
    return pl.pallas_call(
        _gmm_body,
        grid_spec=grid_spec,
        out_shape=jax.ShapeDtypeStruct((P, C), jnp.float32),
    )(bexp, bvalid, xs,
      We1.astype(jnp.bfloat16), be1.reshape(E, 1, FF),
      We2.astype(jnp.bfloat16), be2.reshape(E, 1, C))


def _add_body(a_ref, b_ref, o_ref):
    o_ref[...] = a_ref[...] + b_ref[...]


def _add(a, b):
    return pl.pallas_call(
        _add_body,
        grid=(B,),
        in_specs=[
            pl.BlockSpec((1, T, C), lambda i: (i, 0, 0)),
            pl.BlockSpec((1, T, C), lambda i: (i, 0, 0)),
        ],
        out_specs=pl.BlockSpec((1, T, C), lambda i: (i, 0, 0)),
        out_shape=jax.ShapeDtypeStruct((B, T, C), jnp.float32),
    )(a, b)


def kernel(x, ln1_g, ln1_b, Wq, Wk, Wv, Wp, bp, ln2_g, ln2_b, Wg, We1, be1, We2, be2):
    q, k, v = _qkv(x, ln1_g, ln1_b, Wq, Wk, Wv)
    att = _attn(q, k, v)
    x1, h2, sel3 = _post(x, att, Wp, bp, ln2_g, ln2_b, Wg)

    sel = sel3.reshape(N)
    onehot = (sel[:, None] == jnp.arange(E, dtype=jnp.int32)[None, :])
    cum = jnp.cumsum(onehot.astype(jnp.int32), axis=0)
    rank = jnp.take_along_axis(cum, sel[:, None], axis=1)[:, 0] - 1
    counts = cum[-1]
    pc = ((counts + BM - 1) // BM) * BM
    blocks_cum = jnp.cumsum(pc // BM).astype(jnp.int32)
    start_p = jnp.concatenate([jnp.zeros((1,), jnp.int32),
                               (jnp.cumsum(pc)[:-1]).astype(jnp.int32)])
    g_used = blocks_cum[-1]
    gar = jnp.arange(G, dtype=jnp.int32)
    bexp = jnp.searchsorted(blocks_cum, gar, side='right').astype(jnp.int32)
    bvalid = (gar < g_used).astype(jnp.int32)
    last_e = bexp[jnp.maximum(g_used - 1, 0)]
    bexp = jnp.where(bvalid == 1, bexp, last_e)
    ppos = start_p[sel] + rank
    ridx = (jnp.arange(P, dtype=jnp.int32) % N).at[ppos].set(
        jnp.arange(N, dtype=jnp.int32))

    xs = _sc_gather(h2.reshape(N, C), ridx, P, P // 32)
    ys = _gmm(bexp, bvalid, xs, We1, be1, We2, be2)
    res = _sc_gather(ys, ppos, N, N // 32)
    return _add(x1, res.reshape(B, T, C))


# speedup vs baseline: 1.2323x; 1.0828x over previous
import functools

import jax
import jax.numpy as jnp
from jax import lax
from jax.experimental import pallas as pl
from jax.experimental.pallas import tpu as pltpu
from jax.experimental.pallas import tpu_sc as plsc

B, T, C = 4, 2048, 384
H, D = 6, 64
E = 8
FF = 4 * C
N = B * T

BQ = 512
BM = 256
G = N // BM + E
P = G * BM

_NEG = -1e30


def _ln(xx, g, b):
    m = jnp.mean(xx, axis=-1, keepdims=True)
    v = jnp.mean((xx - m) ** 2, axis=-1, keepdims=True)
    return (xx - m) * lax.rsqrt(v + 1e-5) * g + b


def _qkv_body(x_ref, g_ref, b_ref, w_ref, q_ref, k_ref, v_ref):
    hh = _ln(x_ref[0], g_ref[...], b_ref[...])
    qkv = jnp.dot(hh.astype(jnp.bfloat16), w_ref[...],
                  preferred_element_type=jnp.float32)
    for h in range(H):
        q_ref[0, h] = (qkv[:, h * D:(h + 1) * D] * (C ** -0.5)).astype(jnp.bfloat16)
        k_ref[0, h] = qkv[:, (H + h) * D:(H + h + 1) * D].astype(jnp.bfloat16)
        v_ref[0, h] = qkv[:, (2 * H + h) * D:(2 * H + h + 1) * D].astype(jnp.bfloat16)


def _qkv(x, ln1_g, ln1_b, Wq, Wk, Wv):
    Wqkv = jnp.concatenate([
        Wq.transpose(1, 0, 2).reshape(C, H * D),
        Wk.transpose(1, 0, 2).reshape(C, H * D),
        Wv.transpose(1, 0, 2).reshape(C, H * D),
    ], axis=1).astype(jnp.bfloat16)
    out = jax.ShapeDtypeStruct((B, H, T, D), jnp.bfloat16)
    return pl.pallas_call(
        _qkv_body,
        grid=(B,),
        in_specs=[
            pl.BlockSpec((1, T, C), lambda b: (b, 0, 0)),
            pl.BlockSpec((1, C), lambda b: (0, 0)),
            pl.BlockSpec((1, C), lambda b: (0, 0)),
            pl.BlockSpec((C, 3 * H * D), lambda b: (0, 0)),
        ],
        out_specs=[
            pl.BlockSpec((1, H, T, D), lambda b: (b, 0, 0, 0)),
            pl.BlockSpec((1, H, T, D), lambda b: (b, 0, 0, 0)),
            pl.BlockSpec((1, H, T, D), lambda b: (b, 0, 0, 0)),
        ],
        out_shape=[out, out, out],
    )(x, ln1_g.reshape(1, C), ln1_b.reshape(1, C), Wqkv)


def _attn_body(q_ref, k_ref, v_ref, o_ref):
    i = pl.program_id(2)
    q = q_ref[0, 0]
    tri = (lax.broadcasted_iota(jnp.int32, (BQ, BQ), 1)
           <= lax.broadcasted_iota(jnp.int32, (BQ, BQ), 0))

    def step(j, carry, masked):
        m, l, acc = carry
        kj = k_ref[0, 0, pl.ds(j * BQ, BQ), :]
        s = lax.dot_general(q, kj, (((1,), (1,)), ((), ())),
                            preferred_element_type=jnp.float32)
        if masked:
            s = jnp.where(tri, s, _NEG)
        mn = jnp.maximum(m, jnp.max(s, axis=-1, keepdims=True))
        p = jnp.exp(s - mn)
        alpha = jnp.exp(m - mn)
        l = l * alpha + jnp.sum(p, axis=-1, keepdims=True)
        vj = v_ref[0, 0, pl.ds(j * BQ, BQ), :]
        acc = acc * alpha + jnp.dot(p.astype(jnp.bfloat16), vj,
                                    preferred_element_type=jnp.float32)
        return mn, l, acc

    m0 = jnp.full((BQ, 1), _NEG, jnp.float32)
    l0 = jnp.zeros((BQ, 1), jnp.float32)
    a0 = jnp.zeros((BQ, D), jnp.float32)
    carry = lax.fori_loop(0, i, lambda j, c: step(j, c, False), (m0, l0, a0))
    m, l, acc = step(i, carry, True)
    o_ref[0, 0] = (acc / l).astype(jnp.bfloat16)


def _attn(q, k, v):
    return pl.pallas_call(
        _attn_body,
        grid=(B, H, T // BQ),
        in_specs=[
            pl.BlockSpec((1, 1, BQ, D), lambda b, h, i: (b, h, i, 0)),
            pl.BlockSpec((1, 1, T, D), lambda b, h, i: (b, h, 0, 0)),
            pl.BlockSpec((1, 1, T, D), lambda b, h, i: (b, h, 0, 0)),
        ],
        out_specs=pl.BlockSpec((1, 1, BQ, D), lambda b, h, i: (b, h, i, 0)),
        out_shape=jax.ShapeDtypeStruct((B, H, T, D), jnp.bfloat16),
    )(q, k, v)


def _post_body(x_ref, att_ref, wp_ref, bp_ref, g_ref, b_ref, wg_ref,
               x1_ref, h2_ref, sel_ref):
    acc = x_ref[0] + bp_ref[...]
    for h in range(H):
        acc = acc + jnp.dot(att_ref[0, h],
                            wp_ref[h * D:(h + 1) * D, :].astype(jnp.bfloat16),
                            preferred_element_type=jnp.float32)
    x1_ref[0] = acc
    h2 = _ln(acc, g_ref[...], b_ref[...])
    h2_ref[0] = h2
    logits = jnp.dot(h2, wg_ref[...], preferred_element_type=jnp.float32)
    best_v = logits[:, 0:1]
    best_i = jnp.zeros((T, 1), jnp.int32)
    for e in range(1, E):
        ve = logits[:, e:e + 1]
        upd = ve > best_v
        best_v = jnp.where(upd, ve, best_v)
        best_i = jnp.where(upd, e, best_i)
    sel_ref[0] = best_i


def _post(x, att, Wp, bp, ln2_g, ln2_b, Wg):
    return pl.pallas_call(
        _post_body,
        grid=(B,),
        in_specs=[
            pl.BlockSpec((1, T, C), lambda b: (b, 0, 0)),
            pl.BlockSpec((1, H, T, D), lambda b: (b, 0, 0, 0)),
            pl.BlockSpec((C, C), lambda b: (0, 0)),
            pl.BlockSpec((1, C), lambda b: (0, 0)),
            pl.BlockSpec((1, C), lambda b: (0, 0)),
            pl.BlockSpec((1, C), lambda b: (0, 0)),
            pl.BlockSpec((C, E), lambda b: (0, 0)),
        ],
        out_specs=[
            pl.BlockSpec((1, T, C), lambda b: (b, 0, 0)),
            pl.BlockSpec((1, T, C), lambda b: (b, 0, 0)),
            pl.BlockSpec((1, T, 1), lambda b: (b, 0, 0)),
        ],
        out_shape=[
            jax.ShapeDtypeStruct((B, T, C), jnp.float32),
            jax.ShapeDtypeStruct((B, T, C), jnp.float32),
            jax.ShapeDtypeStruct((B, T, 1), jnp.int32),
        ],
    )(x, att, Wp, bp.reshape(1, C), ln2_g.reshape(1, C), ln2_b.reshape(1, C), Wg)


def _sc_gather(table, idx, rows, chunk):
    nw = 32
    b_per_w = rows // nw
    n_ch = b_per_w // chunk
    mesh = plsc.VectorSubcoreMesh(core_axis_name="c", subcore_axis_name="s")

    @functools.partial(
        pl.kernel,
        out_type=jax.ShapeDtypeStruct((rows, C), jnp.float32),
        mesh=mesh,
        scratch_types=[
            pltpu.VMEM((chunk,), jnp.int32),
            pltpu.VMEM((chunk, C), jnp.float32),
            pltpu.SemaphoreType.DMA,
        ],
    )
    def k(table_hbm, idx_hbm, out_hbm, idx_v, rows_v, sem):
        wid = lax.axis_index("s") * 2 + lax.axis_index("c")
        base0 = wid * b_per_w
        for i in range(n_ch):
            base = base0 + i * chunk
            pltpu.sync_copy(idx_hbm.at[pl.ds(base, chunk)], idx_v)
            pltpu.async_copy(table_hbm.at[idx_v], rows_v, sem).wait()
            pltpu.sync_copy(rows_v, out_hbm.at[pl.ds(base, chunk)])

    return k(table, idx)


def _gmm_body(be_ref, bv_ref, xs_ref, w1_ref, b1_ref, w2_ref, b2_ref, out_ref):
    i = pl.program_id(0)

    @pl.when(bv_ref[i] == 1)
    def _():
        e = be_ref[i]
        a = jnp.dot(xs_ref[...].astype(jnp.bfloat16),
                    w1_ref[e].astype(jnp.bfloat16),
                    preferred_element_type=jnp.float32)
        a = jnp.maximum(a + b1_ref[e], 0.0)
        out_ref[...] = (jnp.dot(a.astype(jnp.bfloat16),
                                w2_ref[e].astype(jnp.bfloat16),
                                preferred_element_type=jnp.float32)
                        + b2_ref[e])


def _gmm(bexp, bvalid, xs, We1, be1, We2, be2):
    grid_spec = pltpu.PrefetchScalarGridSpec(
        num_scalar_prefetch=2,
        grid=(G,),
        in_specs=[
            pl.BlockSpec((BM, C), lambda i, be, bv: (i, 0)),
            pl.BlockSpec((E, C, FF), lambda i, be, bv: (0, 0, 0)),
            pl.BlockSpec((E, 1, FF), lambda i, be, bv: (0, 0, 0)),
            pl.BlockSpec((E, FF, C), lambda i, be, bv: (0, 0, 0)),
            pl.BlockSpec((E, 1, C), lambda i, be, bv: (0, 0, 0)),
        ],
        out_specs=pl.BlockSpec((BM, C), lambda i, be, bv: (i, 0)),
    )
    return pl.pallas_call(
        _gmm_body,
        grid_spec=grid_spec,
        out_shape=jax.ShapeDtypeStruct((P, C), jnp.float32),
    )(bexp, bvalid, xs,
      We1, be1.reshape(E, 1, FF),
      We2, be2.reshape(E, 1, C))


def _add_body(a_ref, b_ref, o_ref):
    o_ref[...] = a_ref[...] + b_ref[...]


def _add(a, b):
    return pl.pallas_call(
        _add_body,
        grid=(B,),
        in_specs=[
            pl.BlockSpec((1, T, C), lambda i: (i, 0, 0)),
            pl.BlockSpec((1, T, C), lambda i: (i, 0, 0)),
        ],
        out_specs=pl.BlockSpec((1, T, C), lambda i: (i, 0, 0)),
        out_shape=jax.ShapeDtypeStruct((B, T, C), jnp.float32),
    )(a, b)


def kernel(x, ln1_g, ln1_b, Wq, Wk, Wv, Wp, bp, ln2_g, ln2_b, Wg, We1, be1, We2, be2):
    q, k, v = _qkv(x, ln1_g, ln1_b, Wq, Wk, Wv)
    att = _attn(q, k, v)
    x1, h2, sel3 = _post(x, att, Wp, bp, ln2_g, ln2_b, Wg)

    sel = sel3.reshape(N)
    onehot = (sel[:, None] == jnp.arange(E, dtype=jnp.int32)[None, :])
    cum = jnp.cumsum(onehot.astype(jnp.int32), axis=0)
    rank = jnp.take_along_axis(cum, sel[:, None], axis=1)[:, 0] - 1
    counts = cum[-1]
    pc = ((counts + BM - 1) // BM) * BM
    blocks_cum = jnp.cumsum(pc // BM).astype(jnp.int32)
    start_p = jnp.concatenate([jnp.zeros((1,), jnp.int32),
                               (jnp.cumsum(pc)[:-1]).astype(jnp.int32)])
    g_used = blocks_cum[-1]
    gar = jnp.arange(G, dtype=jnp.int32)
    bexp = jnp.searchsorted(blocks_cum, gar, side='right').astype(jnp.int32)
    bvalid = (gar < g_used).astype(jnp.int32)
    last_e = bexp[jnp.maximum(g_used - 1, 0)]
    bexp = jnp.where(bvalid == 1, bexp, last_e)
    ppos = start_p[sel] + rank
    ridx = (jnp.arange(P, dtype=jnp.int32) % N).at[ppos].set(
        jnp.arange(N, dtype=jnp.int32))

    xs = _sc_gather(h2.reshape(N, C), ridx, P, P // 32)
    ys = _gmm(bexp, bvalid, xs, We1, be1, We2, be2)
    res = _sc_gather(ys, ppos, N, N // 32)
    return _add(x1, res.reshape(B, T, C))
